# Initial kernel scaffold; baseline (speedup 1.0000x reference)
import jax
import jax.numpy as jnp
from jax.experimental import pallas as pl


def kernel(token_table, type_table, height_table, W_t, W_l, b_conv, central_table, A, gru_W, gru_U, w_attn, fc_W, fc_b, ast_node_token_id, batch_node_type_id, batch_node_sub_tokens_id, batch_node_height, batch_children_index, ast_node_index, batch_tree_index, edge_index, edge_type, in_degrees, segment_ids, last_stmts):
    raise NotImplementedError("write your pallas kernel here")


# speedup vs baseline: 4.7932x; 4.7932x over previous
import functools

import jax
import jax.numpy as jnp
from jax.experimental import pallas as pl
from jax.experimental.pallas import tpu as pltpu

D = 128
N_AST = 8000
N_TREE = 2000
N = N_AST + N_TREE
T = 32
C = 4
E = 320000
N_ETYPES = 3
G = 100
N_CLASSES = 104
TIME_STEPS = [3, 3]

TB = 8
RB = 1000


def _tbcnn_body(feats_ref, child_ref, wt_ref, wl_ref, b_ref, out_ref):
    feats = feats_ref[...]
    child = child_ref[...]
    iota = jax.lax.broadcasted_iota(jnp.int32, (TB, T, C, T), 3)
    onehot = (child[..., None] == iota).astype(jnp.float32)
    p = jnp.sum(onehot, axis=2) * (1.0 / C)
    mean_child = jax.lax.dot_general(
        p, feats, (((2,), (1,)), ((0,), (0,))),
        preferred_element_type=jnp.float32)
    f2 = feats.reshape(TB * T, D)
    mc2 = mean_child.reshape(TB * T, D)
    conv = jnp.maximum(
        jnp.dot(f2, wt_ref[...], preferred_element_type=jnp.float32)
        + jnp.dot(mc2, wl_ref[...], preferred_element_type=jnp.float32)
        + b_ref[...], 0.0)
    out_ref[...] = jnp.max(conv.reshape(TB, T, D), axis=1)


def _tbcnn(feats, children, w_t, w_l, b_conv):
    return pl.pallas_call(
        _tbcnn_body,
        grid=(N_TREE // TB,),
        in_specs=[
            pl.BlockSpec((TB, T, D), lambda i: (i, 0, 0)),
            pl.BlockSpec((TB, T, C), lambda i: (i, 0, 0)),
            pl.BlockSpec((D, D), lambda i: (0, 0)),
            pl.BlockSpec((D, D), lambda i: (0, 0)),
            pl.BlockSpec((1, D), lambda i: (0, 0)),
        ],
        out_specs=pl.BlockSpec((TB, D), lambda i: (i, 0)),
        out_shape=jax.ShapeDtypeStruct((N_TREE, D), jnp.float32),
    )(feats, children, w_t, w_l, b_conv.reshape(1, D))


def _transform_body(h_ref, a_ref, tf_ref):
    h = h_ref[...]
    for e in range(N_ETYPES):
        tf_ref[e] = jnp.dot(h, a_ref[e], preferred_element_type=jnp.float32)


def _transform(h, a_l):
    return pl.pallas_call(
        _transform_body,
        grid=(N // RB,),
        in_specs=[
            pl.BlockSpec((RB, D), lambda i: (i, 0)),
            pl.BlockSpec((N_ETYPES, D, D), lambda i: (0, 0, 0)),
        ],
        out_specs=pl.BlockSpec((N_ETYPES, RB, D), lambda i: (0, i, 0)),
        out_shape=jax.ShapeDtypeStruct((N_ETYPES, N, D), jnp.float32),
    )(h, a_l)


def _step_body(do_ln, with_tf, h_ref, agg_ref, gw_ref, gu_ref, an_ref,
               hn_ref, tf_ref=None):
    h = h_ref[...]
    agg = agg_ref[...]
    z = jax.nn.sigmoid(
        jnp.dot(agg, gw_ref[0], preferred_element_type=jnp.float32)
        + jnp.dot(h, gu_ref[0], preferred_element_type=jnp.float32))
    r = jax.nn.sigmoid(
        jnp.dot(agg, gw_ref[1], preferred_element_type=jnp.float32)
        + jnp.dot(h, gu_ref[1], preferred_element_type=jnp.float32))
    hh = jnp.tanh(
        jnp.dot(agg, gw_ref[2], preferred_element_type=jnp.float32)
        + jnp.dot(r * h, gu_ref[2], preferred_element_type=jnp.float32))
    hn = (1.0 - z) * h + z * hh
    if do_ln:
        m = jnp.mean(hn, axis=-1, keepdims=True)
        v = jnp.mean((hn - m) ** 2, axis=-1, keepdims=True)
        hn = (hn - m) * jax.lax.rsqrt(v + 1e-5)
    hn_ref[...] = hn
    if with_tf:
        for e in range(N_ETYPES):
            tf_ref[e] = jnp.dot(hn, an_ref[e], preferred_element_type=jnp.float32)


def _ggnn_step(h, agg, gw_l, gu_l, a_next, do_ln, with_tf):
    in_specs = [
        pl.BlockSpec((RB, D), lambda i: (i, 0)),
        pl.BlockSpec((RB, D), lambda i: (i, 0)),
        pl.BlockSpec((N_ETYPES, D, D), lambda i: (0, 0, 0)),
        pl.BlockSpec((N_ETYPES, D, D), lambda i: (0, 0, 0)),
        pl.BlockSpec((N_ETYPES, D, D), lambda i: (0, 0, 0)),
    ]
    if with_tf:
        out_specs = [
            pl.BlockSpec((RB, D), lambda i: (i, 0)),
            pl.BlockSpec((N_ETYPES, RB, D), lambda i: (0, i, 0)),
        ]
        out_shape = [
            jax.ShapeDtypeStruct((N, D), jnp.float32),
            jax.ShapeDtypeStruct((N_ETYPES, N, D), jnp.float32),
        ]
    else:
        out_specs = pl.BlockSpec((RB, D), lambda i: (i, 0))
        out_shape = jax.ShapeDtypeStruct((N, D), jnp.float32)
    return pl.pallas_call(
        functools.partial(_step_body, do_ln, with_tf),
        grid=(N // RB,),
        in_specs=in_specs,
        out_specs=out_specs,
        out_shape=out_shape,
    )(h, agg, gw_l, gu_l, a_next)


def _pool_body(h_ref, hl_ref, wa_ref, seg_ref, fcw_ref, fcb_ref, out_ref):
    ctx = jnp.dot(hl_ref[...], wa_ref[...],
                  preferred_element_type=jnp.float32)
    seg = seg_ref[...]
    iota = jax.lax.broadcasted_iota(jnp.int32, (N, 128), 1)
    onehot = (seg == iota).astype(jnp.float32)
    ctx_rows = jnp.dot(onehot, ctx, preferred_element_type=jnp.float32)
    h = h_ref[...]
    scores = jax.nn.sigmoid(
        jnp.sum(h * ctx_rows, axis=-1, keepdims=True))
    ge = jax.lax.dot_general(
        onehot, scores * h, (((0,), (0,)), ((), ())),
        preferred_element_type=jnp.float32)
    out_ref[...] = jnp.dot(ge, fcw_ref[...],
                           preferred_element_type=jnp.float32) + fcb_ref[...]


def _pool(h, hl_pad, w_attn, segment_ids, fc_w_pad, fc_b_pad):
    return pl.pallas_call(
        _pool_body,
        in_specs=[
            pl.BlockSpec((N, D), lambda: (0, 0)),
            pl.BlockSpec((128, D), lambda: (0, 0)),
            pl.BlockSpec((D, D), lambda: (0, 0)),
            pl.BlockSpec((N, 1), lambda: (0, 0)),
            pl.BlockSpec((D, 128), lambda: (0, 0)),
            pl.BlockSpec((1, 128), lambda: (0, 0)),
        ],
        out_specs=pl.BlockSpec((128, 128), lambda: (0, 0)),
        out_shape=jax.ShapeDtypeStruct((128, 128), jnp.float32),
    )(h, hl_pad, w_attn, segment_ids.reshape(N, 1), fc_w_pad, fc_b_pad)


def kernel(token_table, type_table, height_table, W_t, W_l, b_conv,
           central_table, A, gru_W, gru_U, w_attn, fc_W, fc_b,
           ast_node_token_id, batch_node_type_id, batch_node_sub_tokens_id,
           batch_node_height, batch_children_index, ast_node_index,
           batch_tree_index, edge_index, edge_type, in_degrees, segment_ids,
           last_stmts):
    ast_emb = token_table[ast_node_token_id]
    feats = (type_table[batch_node_type_id]
             + token_table[batch_node_sub_tokens_id]
             + height_table[jnp.clip(batch_node_height, 0, 29)])
    tree_emb = _tbcnn(feats, batch_children_index, W_t, W_l, b_conv)
    embeddings = jnp.concatenate([ast_emb, tree_emb], axis=0)
    permcat = jnp.concatenate([ast_node_index, batch_tree_index])
    h = (jnp.zeros((N, D), jnp.float32).at[permcat].set(embeddings)
         + central_table[jnp.clip(in_degrees, 0, 149)])

    src, dst = edge_index[0], edge_index[1]
    gidx = edge_type * N + src
    layer_of_step = [l for l in range(len(TIME_STEPS))
                     for _ in range(TIME_STEPS[l])]
    n_steps = len(layer_of_step)
    tf = _transform(h, A[layer_of_step[0]])
    for i, l in enumerate(layer_of_step):
        msg = tf.reshape(N_ETYPES * N, D)[gidx]
        agg = jax.ops.segment_sum(msg, dst, num_segments=N)
        do_ln = (i + 1 == n_steps) or (layer_of_step[i + 1] != l)
        with_tf = i + 1 < n_steps
        l_next = layer_of_step[min(i + 1, n_steps - 1)]
        res = _ggnn_step(h, agg, gru_W[l], gru_U[l], A[l_next], do_ln, with_tf)
        if with_tf:
            h, tf = res
        else:
            h = res

    hl_pad = jnp.zeros((128, D), jnp.float32).at[:G].set(h[last_stmts])
    fc_w_pad = jnp.zeros((D, 128), jnp.float32).at[:, :N_CLASSES].set(fc_W)
    fc_b_pad = jnp.zeros((1, 128), jnp.float32).at[0, :N_CLASSES].set(fc_b)
    logits = _pool(h, hl_pad, w_attn, segment_ids, fc_w_pad, fc_b_pad)
    return (embeddings, logits[:G, :N_CLASSES])


# speedup vs baseline: 15.8261x; 3.3018x over previous
import functools

import jax
import jax.numpy as jnp
from jax import lax
from jax.experimental import pallas as pl
from jax.experimental.pallas import tpu as pltpu
from jax.experimental.pallas import tpu_sc as plsc

D = 128
N_AST = 8000
N_TREE = 2000
N = N_AST + N_TREE
T = 32
C = 4
E = 320000
N_ETYPES = 3
G = 100
N_CLASSES = 104
TIME_STEPS = [3, 3]

TB = 8
RB = 1000


def _tbcnn_body(feats_ref, child_ref, wt_ref, wl_ref, b_ref, out_ref):
    feats = feats_ref[...]
    child = child_ref[...]
    iota = jax.lax.broadcasted_iota(jnp.int32, (TB, T, C, T), 3)
    onehot = (child[..., None] == iota).astype(jnp.float32)
    p = jnp.sum(onehot, axis=2) * (1.0 / C)
    mean_child = jax.lax.dot_general(
        p, feats, (((2,), (1,)), ((0,), (0,))),
        preferred_element_type=jnp.float32)
    f2 = feats.reshape(TB * T, D)
    mc2 = mean_child.reshape(TB * T, D)
    conv = jnp.maximum(
        jnp.dot(f2, wt_ref[...], preferred_element_type=jnp.float32)
        + jnp.dot(mc2, wl_ref[...], preferred_element_type=jnp.float32)
        + b_ref[...], 0.0)
    out_ref[...] = jnp.max(conv.reshape(TB, T, D), axis=1)


def _tbcnn(feats, children, w_t, w_l, b_conv):
    return pl.pallas_call(
        _tbcnn_body,
        grid=(N_TREE // TB,),
        in_specs=[
            pl.BlockSpec((TB, T, D), lambda i: (i, 0, 0)),
            pl.BlockSpec((TB, T, C), lambda i: (i, 0, 0)),
            pl.BlockSpec((D, D), lambda i: (0, 0)),
            pl.BlockSpec((D, D), lambda i: (0, 0)),
            pl.BlockSpec((1, D), lambda i: (0, 0)),
        ],
        out_specs=pl.BlockSpec((TB, D), lambda i: (i, 0)),
        out_shape=jax.ShapeDtypeStruct((N_TREE, D), jnp.float32),
    )(feats, children, w_t, w_l, b_conv.reshape(1, D))


def _transform_body(h_ref, a_ref, tf_ref):
    h = h_ref[...]
    for e in range(N_ETYPES):
        tf_ref[e] = jnp.dot(h, a_ref[e], preferred_element_type=jnp.float32)


def _transform(h, a_l):
    return pl.pallas_call(
        _transform_body,
        grid=(N // RB,),
        in_specs=[
            pl.BlockSpec((RB, D), lambda i: (i, 0)),
            pl.BlockSpec((N_ETYPES, D, D), lambda i: (0, 0, 0)),
        ],
        out_specs=pl.BlockSpec((N_ETYPES, RB, D), lambda i: (0, i, 0)),
        out_shape=jax.ShapeDtypeStruct((N_ETYPES, N, D), jnp.float32),
    )(h, a_l)


def _step_body(do_ln, with_tf, h_ref, agg_ref, gw_ref, gu_ref, an_ref,
               hn_ref, tf_ref=None):
    h = h_ref[...]
    agg = agg_ref[0] + agg_ref[1]
    z = jax.nn.sigmoid(
        jnp.dot(agg, gw_ref[0], preferred_element_type=jnp.float32)
        + jnp.dot(h, gu_ref[0], preferred_element_type=jnp.float32))
    r = jax.nn.sigmoid(
        jnp.dot(agg, gw_ref[1], preferred_element_type=jnp.float32)
        + jnp.dot(h, gu_ref[1], preferred_element_type=jnp.float32))
    hh = jnp.tanh(
        jnp.dot(agg, gw_ref[2], preferred_element_type=jnp.float32)
        + jnp.dot(r * h, gu_ref[2], preferred_element_type=jnp.float32))
    hn = (1.0 - z) * h + z * hh
    if do_ln:
        m = jnp.mean(hn, axis=-1, keepdims=True)
        v = jnp.mean((hn - m) ** 2, axis=-1, keepdims=True)
        hn = (hn - m) * jax.lax.rsqrt(v + 1e-5)
    hn_ref[...] = hn
    if with_tf:
        for e in range(N_ETYPES):
            tf_ref[e] = jnp.dot(hn, an_ref[e], preferred_element_type=jnp.float32)


def _ggnn_step(h, agg, gw_l, gu_l, a_next, do_ln, with_tf):
    in_specs = [
        pl.BlockSpec((RB, D), lambda i: (i, 0)),
        pl.BlockSpec((SC_CORES, RB, D), lambda i: (0, i, 0)),
        pl.BlockSpec((N_ETYPES, D, D), lambda i: (0, 0, 0)),
        pl.BlockSpec((N_ETYPES, D, D), lambda i: (0, 0, 0)),
        pl.BlockSpec((N_ETYPES, D, D), lambda i: (0, 0, 0)),
    ]
    if with_tf:
        out_specs = [
            pl.BlockSpec((RB, D), lambda i: (i, 0)),
            pl.BlockSpec((N_ETYPES, RB, D), lambda i: (0, i, 0)),
        ]
        out_shape = [
            jax.ShapeDtypeStruct((N, D), jnp.float32),
            jax.ShapeDtypeStruct((N_ETYPES, N, D), jnp.float32),
        ]
    else:
        out_specs = pl.BlockSpec((RB, D), lambda i: (i, 0))
        out_shape = jax.ShapeDtypeStruct((N, D), jnp.float32)
    return pl.pallas_call(
        functools.partial(_step_body, do_ln, with_tf),
        grid=(N // RB,),
        in_specs=in_specs,
        out_specs=out_specs,
        out_shape=out_shape,
    )(h, agg, gw_l, gu_l, a_next)


SC_CORES = 2
SC_SUBCORES = 16
NW = SC_CORES * SC_SUBCORES
EPW = E // NW
K = 80
NITER = EPW // K
N_PAD = 10240
RPS = N_PAD // SC_SUBCORES
ZR = 128

_sc_mesh = plsc.VectorSubcoreMesh(core_axis_name="c", subcore_axis_name="s")


@functools.partial(
    pl.kernel,
    out_type=jax.ShapeDtypeStruct((SC_CORES, N_PAD, D), jnp.float32),
    mesh=_sc_mesh,
    scratch_types=[
        pltpu.VMEM((K,), jnp.int32),
        pltpu.VMEM((K,), jnp.int32),
        pltpu.VMEM((K, D), jnp.float32),
        pltpu.VMEM((ZR, D), jnp.float32),
        pltpu.VMEM_SHARED((N_PAD, D), jnp.float32),
        pltpu.SemaphoreType.DMA,
    ],
)
def _edge_agg(tf_hbm, gidx_hbm, dst_hbm, out_hbm,
              idx_v, dst_v, rows_v, zero_v, agg_sh, sem):
    c = lax.axis_index("c")
    s = lax.axis_index("s")
    wid = c * SC_SUBCORES + s

    @pl.loop(0, ZR)
    def _zero_rows(i):
        for j in range(D // 16):
            zero_v[i, pl.ds(j * 16, 16)] = jnp.zeros((16,), jnp.float32)

    row0 = s * RPS
    for b in range(RPS // ZR):
        pltpu.sync_copy(zero_v, agg_sh.at[pl.ds(row0 + b * ZR, ZR), :])
    plsc.subcore_barrier()

    ebase = wid * EPW

    @pl.loop(0, NITER)
    def _edges(it):
        off = ebase + it * K
        pltpu.sync_copy(gidx_hbm.at[pl.ds(off, K)], idx_v)
        pltpu.sync_copy(dst_hbm.at[pl.ds(off, K)], dst_v)
        pltpu.async_copy(tf_hbm.at[idx_v], rows_v, sem).wait()
        pltpu.sync_copy(rows_v, agg_sh.at[dst_v], add=True)

    plsc.subcore_barrier()
    pltpu.sync_copy(agg_sh.at[pl.ds(row0, RPS), :],
                    out_hbm.at[c, pl.ds(row0, RPS), :])


def _pool_body(h_ref, hl_ref, wa_ref, seg_ref, fcw_ref, fcb_ref, out_ref):
    ctx = jnp.dot(hl_ref[...], wa_ref[...],
                  preferred_element_type=jnp.float32)
    seg = seg_ref[...]
    iota = jax.lax.broadcasted_iota(jnp.int32, (N, 128), 1)
    onehot = (seg == iota).astype(jnp.float32)
    ctx_rows = jnp.dot(onehot, ctx, preferred_element_type=jnp.float32)
    h = h_ref[...]
    scores = jax.nn.sigmoid(
        jnp.sum(h * ctx_rows, axis=-1, keepdims=True))
    ge = jax.lax.dot_general(
        onehot, scores * h, (((0,), (0,)), ((), ())),
        preferred_element_type=jnp.float32)
    out_ref[...] = jnp.dot(ge, fcw_ref[...],
                           preferred_element_type=jnp.float32) + fcb_ref[...]


def _pool(h, hl_pad, w_attn, segment_ids, fc_w_pad, fc_b_pad):
    return pl.pallas_call(
        _pool_body,
        in_specs=[
            pl.BlockSpec((N, D), lambda: (0, 0)),
            pl.BlockSpec((128, D), lambda: (0, 0)),
            pl.BlockSpec((D, D), lambda: (0, 0)),
            pl.BlockSpec((N, 1), lambda: (0, 0)),
            pl.BlockSpec((D, 128), lambda: (0, 0)),
            pl.BlockSpec((1, 128), lambda: (0, 0)),
        ],
        out_specs=pl.BlockSpec((128, 128), lambda: (0, 0)),
        out_shape=jax.ShapeDtypeStruct((128, 128), jnp.float32),
    )(h, hl_pad, w_attn, segment_ids.reshape(N, 1), fc_w_pad, fc_b_pad)


def kernel(token_table, type_table, height_table, W_t, W_l, b_conv,
           central_table, A, gru_W, gru_U, w_attn, fc_W, fc_b,
           ast_node_token_id, batch_node_type_id, batch_node_sub_tokens_id,
           batch_node_height, batch_children_index, ast_node_index,
           batch_tree_index, edge_index, edge_type, in_degrees, segment_ids,
           last_stmts):
    ast_emb = token_table[ast_node_token_id]
    feats = (type_table[batch_node_type_id]
             + token_table[batch_node_sub_tokens_id]
             + height_table[jnp.clip(batch_node_height, 0, 29)])
    tree_emb = _tbcnn(feats, batch_children_index, W_t, W_l, b_conv)
    embeddings = jnp.concatenate([ast_emb, tree_emb], axis=0)
    permcat = jnp.concatenate([ast_node_index, batch_tree_index])
    h = (jnp.zeros((N, D), jnp.float32).at[permcat].set(embeddings)
         + central_table[jnp.clip(in_degrees, 0, 149)])

    src, dst = edge_index[0], edge_index[1]
    gidx = edge_type * N + src
    layer_of_step = [l for l in range(len(TIME_STEPS))
                     for _ in range(TIME_STEPS[l])]
    n_steps = len(layer_of_step)
    tf = _transform(h, A[layer_of_step[0]])
    for i, l in enumerate(layer_of_step):
        agg2 = _edge_agg(tf.reshape(N_ETYPES * N, D), gidx, dst)
        do_ln = (i + 1 == n_steps) or (layer_of_step[i + 1] != l)
        with_tf = i + 1 < n_steps
        l_next = layer_of_step[min(i + 1, n_steps - 1)]
        res = _ggnn_step(h, agg2, gru_W[l], gru_U[l], A[l_next], do_ln, with_tf)
        if with_tf:
            h, tf = res
        else:
            h = res

    hl_pad = jnp.zeros((128, D), jnp.float32).at[:G].set(h[last_stmts])
    fc_w_pad = jnp.zeros((D, 128), jnp.float32).at[:, :N_CLASSES].set(fc_W)
    fc_b_pad = jnp.zeros((1, 128), jnp.float32).at[0, :N_CLASSES].set(fc_b)
    logits = _pool(h, hl_pad, w_attn, segment_ids, fc_w_pad, fc_b_pad)
    return (embeddings, logits[:G, :N_CLASSES])


# speedup vs baseline: 27.4212x; 1.7327x over previous
import functools

import jax
import jax.numpy as jnp
from jax import lax
from jax.experimental import pallas as pl
from jax.experimental.pallas import tpu as pltpu
from jax.experimental.pallas import tpu_sc as plsc

D = 128
N_AST = 8000
N_TREE = 2000
N = N_AST + N_TREE
T = 32
C = 4
E = 320000
N_ETYPES = 3
G = 100
N_CLASSES = 104
TIME_STEPS = [3, 3]

TB = 8
RB = 1000


def _tbcnn_body(feats_ref, child_ref, wt_ref, wl_ref, b_ref, out_ref):
    feats = feats_ref[...]
    child = child_ref[...]
    iota = jax.lax.broadcasted_iota(jnp.int32, (TB, T, C, T), 3)
    onehot = (child[..., None] == iota).astype(jnp.float32)
    p = jnp.sum(onehot, axis=2) * (1.0 / C)
    mean_child = jax.lax.dot_general(
        p, feats, (((2,), (1,)), ((0,), (0,))),
        preferred_element_type=jnp.float32)
    f2 = feats.reshape(TB * T, D)
    mc2 = mean_child.reshape(TB * T, D)
    conv = jnp.maximum(
        jnp.dot(f2, wt_ref[...], preferred_element_type=jnp.float32)
        + jnp.dot(mc2, wl_ref[...], preferred_element_type=jnp.float32)
        + b_ref[...], 0.0)
    out_ref[...] = jnp.max(conv.reshape(TB, T, D), axis=1)


def _tbcnn(feats, children, w_t, w_l, b_conv):
    return pl.pallas_call(
        _tbcnn_body,
        grid=(N_TREE // TB,),
        in_specs=[
            pl.BlockSpec((TB, T, D), lambda i: (i, 0, 0)),
            pl.BlockSpec((TB, T, C), lambda i: (i, 0, 0)),
            pl.BlockSpec((D, D), lambda i: (0, 0)),
            pl.BlockSpec((D, D), lambda i: (0, 0)),
            pl.BlockSpec((1, D), lambda i: (0, 0)),
        ],
        out_specs=pl.BlockSpec((TB, D), lambda i: (i, 0)),
        out_shape=jax.ShapeDtypeStruct((N_TREE, D), jnp.float32),
    )(feats, children, w_t, w_l, b_conv.reshape(1, D))


def _transform_body(h_ref, a_ref, tf_ref):
    h = h_ref[...]
    for e in range(N_ETYPES):
        tf_ref[e] = jnp.dot(h, a_ref[e], preferred_element_type=jnp.float32)


def _transform(h, a_l):
    return pl.pallas_call(
        _transform_body,
        grid=(N // RB,),
        in_specs=[
            pl.BlockSpec((RB, D), lambda i: (i, 0)),
            pl.BlockSpec((N_ETYPES, D, D), lambda i: (0, 0, 0)),
        ],
        out_specs=pl.BlockSpec((N_ETYPES, RB, D), lambda i: (0, i, 0)),
        out_shape=jax.ShapeDtypeStruct((N_ETYPES, N, D), jnp.float32),
    )(h, a_l)


def _step_body(do_ln, with_tf, h_ref, agg_ref, gw_ref, gu_ref, an_ref,
               hn_ref, tf_ref=None):
    h = h_ref[...]
    agg = agg_ref[0] + agg_ref[1]
    z = jax.nn.sigmoid(
        jnp.dot(agg, gw_ref[0], preferred_element_type=jnp.float32)
        + jnp.dot(h, gu_ref[0], preferred_element_type=jnp.float32))
    r = jax.nn.sigmoid(
        jnp.dot(agg, gw_ref[1], preferred_element_type=jnp.float32)
        + jnp.dot(h, gu_ref[1], preferred_element_type=jnp.float32))
    hh = jnp.tanh(
        jnp.dot(agg, gw_ref[2], preferred_element_type=jnp.float32)
        + jnp.dot(r * h, gu_ref[2], preferred_element_type=jnp.float32))
    hn = (1.0 - z) * h + z * hh
    if do_ln:
        m = jnp.mean(hn, axis=-1, keepdims=True)
        v = jnp.mean((hn - m) ** 2, axis=-1, keepdims=True)
        hn = (hn - m) * jax.lax.rsqrt(v + 1e-5)
    hn_ref[...] = hn
    if with_tf:
        for e in range(N_ETYPES):
            tf_ref[e] = jnp.dot(hn, an_ref[e], preferred_element_type=jnp.float32)


def _ggnn_step(h, agg, gw_l, gu_l, a_next, do_ln, with_tf):
    in_specs = [
        pl.BlockSpec((RB, D), lambda i: (i, 0)),
        pl.BlockSpec((SC_CORES, RB, D), lambda i: (0, i, 0)),
        pl.BlockSpec((N_ETYPES, D, D), lambda i: (0, 0, 0)),
        pl.BlockSpec((N_ETYPES, D, D), lambda i: (0, 0, 0)),
        pl.BlockSpec((N_ETYPES, D, D), lambda i: (0, 0, 0)),
    ]
    if with_tf:
        out_specs = [
            pl.BlockSpec((RB, D), lambda i: (i, 0)),
            pl.BlockSpec((N_ETYPES, RB, D), lambda i: (0, i, 0)),
        ]
        out_shape = [
            jax.ShapeDtypeStruct((N, D), jnp.float32),
            jax.ShapeDtypeStruct((N_ETYPES, N, D), jnp.float32),
        ]
    else:
        out_specs = pl.BlockSpec((RB, D), lambda i: (i, 0))
        out_shape = jax.ShapeDtypeStruct((N, D), jnp.float32)
    return pl.pallas_call(
        functools.partial(_step_body, do_ln, with_tf),
        grid=(N // RB,),
        in_specs=in_specs,
        out_specs=out_specs,
        out_shape=out_shape,
    )(h, agg, gw_l, gu_l, a_next)


SC_CORES = 2
SC_SUBCORES = 16
NW = SC_CORES * SC_SUBCORES
EPW = E // NW
K = 80
NITER = EPW // K
N_PAD = 10240
RPS = N_PAD // SC_SUBCORES

_sc_mesh = plsc.VectorSubcoreMesh(core_axis_name="c", subcore_axis_name="s")


@functools.partial(
    pl.kernel,
    out_type=jax.ShapeDtypeStruct((SC_CORES, N_PAD, D), jnp.float32),
    mesh=_sc_mesh,
    scratch_types=[
        pltpu.VMEM((NITER, K), jnp.int32),
        pltpu.VMEM((2, K), jnp.int32),
        pltpu.VMEM((2, K), jnp.int32),
        pltpu.VMEM((K, D), jnp.float32),
        pltpu.VMEM((K, D), jnp.float32),
        pltpu.VMEM_SHARED((N_PAD, D), jnp.float32),
        pltpu.SemaphoreType.DMA,
        pltpu.SemaphoreType.DMA,
        pltpu.SemaphoreType.DMA,
    ],
)
def _edge_agg(tf_hbm, pk_hbm, out_hbm,
              pk_v, idxu, dstu, rows_a, rows_b, agg_sh,
              sem_a, sem_b, sem_i):
    c = lax.axis_index("c")
    s = lax.axis_index("s")
    wid = c * SC_SUBCORES + s
    rows = (rows_a, rows_b)
    sems = (sem_a, sem_b)

    pltpu.async_copy(pk_hbm.at[wid], pk_v, sem_i)

    @pl.loop(0, K)
    def _zero_rows(i):
        for j in range(D // 16):
            rows_a[i, pl.ds(j * 16, 16)] = jnp.zeros((16,), jnp.float32)

    row0 = s * RPS
    for b in range(RPS // K):
        pltpu.sync_copy(rows_a, agg_sh.at[pl.ds(row0 + b * K, K), :])
    pltpu.make_async_copy(pk_hbm.at[wid], pk_v, sem_i).wait()
    plsc.subcore_barrier()

    def _unpack(cur, b):
        for j in range(K // 16):
            v = pk_v[cur, pl.ds(j * 16, 16)]
            idxu[b, pl.ds(j * 16, 16)] = jnp.right_shift(
                v, jnp.full((16,), 14, jnp.int32))
            dstu[b, pl.ds(j * 16, 16)] = jnp.bitwise_and(
                v, jnp.full((16,), 16383, jnp.int32))

    _unpack(0, 0)
    _unpack(1, 1)
    pltpu.async_copy(tf_hbm.at[idxu.at[0]], rows_a, sem_a)
    pltpu.async_copy(tf_hbm.at[idxu.at[1]], rows_b, sem_b)

    @pl.loop(0, 2 * (NITER // 2), step=2)
    def _edges(it):
        for b in range(2):
            cur = it + b
            pltpu.make_async_copy(tf_hbm.at[idxu.at[b]], rows[b],
                                  sems[b]).wait()
            pltpu.sync_copy(rows[b], agg_sh.at[dstu.at[b]], add=True)
            nxt = cur + 2

            @pl.when(nxt < NITER)
            def _prefetch():
                _unpack(nxt, b)
                pltpu.async_copy(tf_hbm.at[idxu.at[b]], rows[b], sems[b])

    if NITER % 2:
        last = NITER - 1
        b = last % 2
        pltpu.make_async_copy(tf_hbm.at[idxu.at[b]], rows[b], sems[b]).wait()
        pltpu.sync_copy(rows[b], agg_sh.at[dstu.at[b]], add=True)

    plsc.subcore_barrier()
    pltpu.sync_copy(agg_sh.at[pl.ds(row0, RPS), :],
                    out_hbm.at[c, pl.ds(row0, RPS), :])


def _pool_body(h_ref, hl_ref, wa_ref, seg_ref, fcw_ref, fcb_ref, out_ref):
    ctx = jnp.dot(hl_ref[...], wa_ref[...],
                  preferred_element_type=jnp.float32)
    seg = seg_ref[...]
    iota = jax.lax.broadcasted_iota(jnp.int32, (N, 128), 1)
    onehot = (seg == iota).astype(jnp.float32)
    ctx_rows = jnp.dot(onehot, ctx, preferred_element_type=jnp.float32)
    h = h_ref[...]
    scores = jax.nn.sigmoid(
        jnp.sum(h * ctx_rows, axis=-1, keepdims=True))
    ge = jax.lax.dot_general(
        onehot, scores * h, (((0,), (0,)), ((), ())),
        preferred_element_type=jnp.float32)
    out_ref[...] = jnp.dot(ge, fcw_ref[...],
                           preferred_element_type=jnp.float32) + fcb_ref[...]


def _pool(h, hl_pad, w_attn, segment_ids, fc_w_pad, fc_b_pad):
    return pl.pallas_call(
        _pool_body,
        in_specs=[
            pl.BlockSpec((N, D), lambda: (0, 0)),
            pl.BlockSpec((128, D), lambda: (0, 0)),
            pl.BlockSpec((D, D), lambda: (0, 0)),
            pl.BlockSpec((N, 1), lambda: (0, 0)),
            pl.BlockSpec((D, 128), lambda: (0, 0)),
            pl.BlockSpec((1, 128), lambda: (0, 0)),
        ],
        out_specs=pl.BlockSpec((128, 128), lambda: (0, 0)),
        out_shape=jax.ShapeDtypeStruct((128, 128), jnp.float32),
    )(h, hl_pad, w_attn, segment_ids.reshape(N, 1), fc_w_pad, fc_b_pad)


def kernel(token_table, type_table, height_table, W_t, W_l, b_conv,
           central_table, A, gru_W, gru_U, w_attn, fc_W, fc_b,
           ast_node_token_id, batch_node_type_id, batch_node_sub_tokens_id,
           batch_node_height, batch_children_index, ast_node_index,
           batch_tree_index, edge_index, edge_type, in_degrees, segment_ids,
           last_stmts):
    ast_emb = token_table[ast_node_token_id]
    feats = (type_table[batch_node_type_id]
             + token_table[batch_node_sub_tokens_id]
             + height_table[jnp.clip(batch_node_height, 0, 29)])
    tree_emb = _tbcnn(feats, batch_children_index, W_t, W_l, b_conv)
    embeddings = jnp.concatenate([ast_emb, tree_emb], axis=0)
    permcat = jnp.concatenate([ast_node_index, batch_tree_index])
    h = (jnp.zeros((N, D), jnp.float32).at[permcat].set(embeddings)
         + central_table[jnp.clip(in_degrees, 0, 149)])

    src, dst = edge_index[0], edge_index[1]
    packed = ((edge_type * N + src) * 16384 + dst).reshape(NW, NITER, K)
    layer_of_step = [l for l in range(len(TIME_STEPS))
                     for _ in range(TIME_STEPS[l])]
    n_steps = len(layer_of_step)
    tf = _transform(h, A[layer_of_step[0]])
    for i, l in enumerate(layer_of_step):
        agg2 = _edge_agg(tf.reshape(N_ETYPES * N, D), packed)
        do_ln = (i + 1 == n_steps) or (layer_of_step[i + 1] != l)
        with_tf = i + 1 < n_steps
        l_next = layer_of_step[min(i + 1, n_steps - 1)]
        res = _ggnn_step(h, agg2, gru_W[l], gru_U[l], A[l_next], do_ln, with_tf)
        if with_tf:
            h, tf = res
        else:
            h = res

    hl_pad = jnp.zeros((128, D), jnp.float32).at[:G].set(h[last_stmts])
    fc_w_pad = jnp.zeros((D, 128), jnp.float32).at[:, :N_CLASSES].set(fc_W)
    fc_b_pad = jnp.zeros((1, 128), jnp.float32).at[0, :N_CLASSES].set(fc_b)
    logits = _pool(h, hl_pad, w_attn, segment_ids, fc_w_pad, fc_b_pad)
    return (embeddings, logits[:G, :N_CLASSES])


# speedup vs baseline: 32.9144x; 1.2003x over previous
import functools

import jax
import jax.numpy as jnp
from jax import lax
from jax.experimental import pallas as pl
from jax.experimental.pallas import tpu as pltpu
from jax.experimental.pallas import tpu_sc as plsc

D = 128
N_AST = 8000
N_TREE = 2000
N = N_AST + N_TREE
T = 32
C = 4
E = 320000
N_ETYPES = 3
G = 100
N_CLASSES = 104
TIME_STEPS = [3, 3]

TB = 8
RB = 1000


def _tbcnn_body(feats_ref, child_ref, wt_ref, wl_ref, b_ref, out_ref):
    feats = feats_ref[...]
    child = child_ref[...]
    iota = jax.lax.broadcasted_iota(jnp.int32, (TB, T, C, T), 3)
    onehot = (child[..., None] == iota).astype(jnp.float32)
    p = jnp.sum(onehot, axis=2) * (1.0 / C)
    mean_child = jax.lax.dot_general(
        p, feats, (((2,), (1,)), ((0,), (0,))),
        preferred_element_type=jnp.float32)
    f2 = feats.reshape(TB * T, D)
    mc2 = mean_child.reshape(TB * T, D)
    conv = jnp.maximum(
        jnp.dot(f2, wt_ref[...], preferred_element_type=jnp.float32)
        + jnp.dot(mc2, wl_ref[...], preferred_element_type=jnp.float32)
        + b_ref[...], 0.0)
    out_ref[...] = jnp.max(conv.reshape(TB, T, D), axis=1)


def _tbcnn(feats, children, w_t, w_l, b_conv):
    return pl.pallas_call(
        _tbcnn_body,
        grid=(N_TREE // TB,),
        in_specs=[
            pl.BlockSpec((TB, T, D), lambda i: (i, 0, 0)),
            pl.BlockSpec((TB, T, C), lambda i: (i, 0, 0)),
            pl.BlockSpec((D, D), lambda i: (0, 0)),
            pl.BlockSpec((D, D), lambda i: (0, 0)),
            pl.BlockSpec((1, D), lambda i: (0, 0)),
        ],
        out_specs=pl.BlockSpec((TB, D), lambda i: (i, 0)),
        out_shape=jax.ShapeDtypeStruct((N_TREE, D), jnp.float32),
    )(feats, children, w_t, w_l, b_conv.reshape(1, D))


def _transform_body(h_ref, a_ref, tf_ref):
    h = h_ref[...]
    for e in range(N_ETYPES):
        tf_ref[e] = jnp.dot(h, a_ref[e], preferred_element_type=jnp.float32)


def _transform(h, a_l):
    return pl.pallas_call(
        _transform_body,
        grid=(N // RB,),
        in_specs=[
            pl.BlockSpec((RB, D), lambda i: (i, 0)),
            pl.BlockSpec((N_ETYPES, D, D), lambda i: (0, 0, 0)),
        ],
        out_specs=pl.BlockSpec((N_ETYPES, RB, D), lambda i: (0, i, 0)),
        out_shape=jax.ShapeDtypeStruct((N_ETYPES, N, D), jnp.float32),
    )(h, a_l)


def _step_body(do_ln, with_tf, h_ref, agg_ref, gw_ref, gu_ref, an_ref,
               hn_ref, tf_ref=None):
    h = h_ref[...]
    agg = agg_ref[0] + agg_ref[1]
    z = jax.nn.sigmoid(
        jnp.dot(agg, gw_ref[0], preferred_element_type=jnp.float32)
        + jnp.dot(h, gu_ref[0], preferred_element_type=jnp.float32))
    r = jax.nn.sigmoid(
        jnp.dot(agg, gw_ref[1], preferred_element_type=jnp.float32)
        + jnp.dot(h, gu_ref[1], preferred_element_type=jnp.float32))
    hh = jnp.tanh(
        jnp.dot(agg, gw_ref[2], preferred_element_type=jnp.float32)
        + jnp.dot(r * h, gu_ref[2], preferred_element_type=jnp.float32))
    hn = (1.0 - z) * h + z * hh
    if do_ln:
        m = jnp.mean(hn, axis=-1, keepdims=True)
        v = jnp.mean((hn - m) ** 2, axis=-1, keepdims=True)
        hn = (hn - m) * jax.lax.rsqrt(v + 1e-5)
    hn_ref[...] = hn
    if with_tf:
        for e in range(N_ETYPES):
            tf_ref[e] = jnp.dot(hn, an_ref[e], preferred_element_type=jnp.float32)


def _ggnn_step(h, agg, gw_l, gu_l, a_next, do_ln, with_tf):
    in_specs = [
        pl.BlockSpec((RB, D), lambda i: (i, 0)),
        pl.BlockSpec((SC_CORES, RB, D), lambda i: (0, i, 0)),
        pl.BlockSpec((N_ETYPES, D, D), lambda i: (0, 0, 0)),
        pl.BlockSpec((N_ETYPES, D, D), lambda i: (0, 0, 0)),
        pl.BlockSpec((N_ETYPES, D, D), lambda i: (0, 0, 0)),
    ]
    if with_tf:
        out_specs = [
            pl.BlockSpec((RB, D), lambda i: (i, 0)),
            pl.BlockSpec((N_ETYPES, RB, D), lambda i: (0, i, 0)),
        ]
        out_shape = [
            jax.ShapeDtypeStruct((N, D), jnp.float32),
            jax.ShapeDtypeStruct((N_ETYPES, N, D), jnp.float32),
        ]
    else:
        out_specs = pl.BlockSpec((RB, D), lambda i: (i, 0))
        out_shape = jax.ShapeDtypeStruct((N, D), jnp.float32)
    return pl.pallas_call(
        functools.partial(_step_body, do_ln, with_tf),
        grid=(N // RB,),
        in_specs=in_specs,
        out_specs=out_specs,
        out_shape=out_shape,
    )(h, agg, gw_l, gu_l, a_next)


SC_CORES = 2
SC_SUBCORES = 16
NW = SC_CORES * SC_SUBCORES
EPW = E // NW
K = 80
NITER = EPW // K
N_PAD = 10240
RPS = N_PAD // SC_SUBCORES

_sc_mesh = plsc.VectorSubcoreMesh(core_axis_name="c", subcore_axis_name="s")


@functools.partial(
    pl.kernel,
    out_type=jax.ShapeDtypeStruct((SC_CORES, N_PAD, D), jnp.float32),
    mesh=_sc_mesh,
    scratch_types=[
        pltpu.VMEM((NITER, K), jnp.int32),
        pltpu.VMEM((2, K), jnp.int32),
        pltpu.VMEM((2, K), jnp.int32),
        pltpu.VMEM((K, D), jnp.float32),
        pltpu.VMEM((K, D), jnp.float32),
        pltpu.VMEM_SHARED((N_PAD, D), jnp.float32),
        pltpu.SemaphoreType.DMA,
        pltpu.SemaphoreType.DMA,
        pltpu.SemaphoreType.DMA,
    ],
)
def _edge_agg(tf_hbm, pk_hbm, out_hbm,
              pk_v, idxu, dstu, rows_a, rows_b, agg_sh,
              sem_a, sem_b, sem_i):
    c = lax.axis_index("c")
    s = lax.axis_index("s")
    wid = c * SC_SUBCORES + s
    rows = (rows_a, rows_b)
    sems = (sem_a, sem_b)

    pltpu.async_copy(pk_hbm.at[wid], pk_v, sem_i)

    @pl.loop(0, K)
    def _zero_rows(i):
        for j in range(D // 16):
            rows_a[i, pl.ds(j * 16, 16)] = jnp.zeros((16,), jnp.float32)

    row0 = s * RPS
    for b in range(RPS // K):
        pltpu.sync_copy(rows_a, agg_sh.at[pl.ds(row0 + b * K, K), :])
    pltpu.make_async_copy(pk_hbm.at[wid], pk_v, sem_i).wait()
    plsc.subcore_barrier()

    def _unpack(cur, b):
        for j in range(K // 16):
            v = pk_v[cur, pl.ds(j * 16, 16)]
            idxu[b, pl.ds(j * 16, 16)] = jnp.right_shift(
                v, jnp.full((16,), 14, jnp.int32))
            dstu[b, pl.ds(j * 16, 16)] = jnp.bitwise_and(
                v, jnp.full((16,), 16383, jnp.int32))

    _unpack(0, 0)
    _unpack(1, 1)
    pltpu.async_copy(tf_hbm.at[idxu.at[0]], rows_a, sem_a)
    pltpu.async_copy(tf_hbm.at[idxu.at[1]], rows_b, sem_b)

    @pl.loop(0, 2 * (NITER // 2), step=2)
    def _edges(it):
        for b in range(2):
            cur = it + b
            pltpu.make_async_copy(tf_hbm.at[idxu.at[b]], rows[b],
                                  sems[b]).wait()
            pltpu.sync_copy(rows[b], agg_sh.at[dstu.at[b]], add=True)
            nxt = cur + 2

            @pl.when(nxt < NITER)
            def _prefetch():
                _unpack(nxt, b)
                pltpu.async_copy(tf_hbm.at[idxu.at[b]], rows[b], sems[b])

    if NITER % 2:
        last = NITER - 1
        b = last % 2
        pltpu.make_async_copy(tf_hbm.at[idxu.at[b]], rows[b], sems[b]).wait()
        pltpu.sync_copy(rows[b], agg_sh.at[dstu.at[b]], add=True)

    plsc.subcore_barrier()
    pltpu.sync_copy(agg_sh.at[pl.ds(row0, RPS), :],
                    out_hbm.at[c, pl.ds(row0, RPS), :])


KF, NF = 80, 25
KA, NA = 64, 4
KH, NH = 80, 4
AST_PAD = NW * KA * NA


def _vadd3(dst, a, b, nrows):
    @pl.loop(0, nrows)
    def _rows(r):
        for j in range(D // 16):
            sl = pl.ds(j * 16, 16)
            dst[r, sl] = dst[r, sl] + a[r, sl] + b[r, sl]


@functools.partial(
    pl.kernel,
    out_type=(jax.ShapeDtypeStruct((N_TREE * T, D), jnp.float32),
              jax.ShapeDtypeStruct((AST_PAD, D), jnp.float32)),
    mesh=_sc_mesh,
    scratch_types=[
        pltpu.VMEM((NF, KF), jnp.int32),
        pltpu.VMEM((NA, KA), jnp.int32),
        pltpu.VMEM((KF,), jnp.int32),
        pltpu.VMEM((KF,), jnp.int32),
        pltpu.VMEM((KF,), jnp.int32),
        pltpu.VMEM((KF, D), jnp.float32),
        pltpu.VMEM((KF, D), jnp.float32),
        pltpu.VMEM((KF, D), jnp.float32),
        pltpu.SemaphoreType.DMA,
        pltpu.SemaphoreType.DMA,
        pltpu.SemaphoreType.DMA,
    ],
)
def _sc_embed(type_hbm, token_hbm, height_hbm, pkf_hbm, pka_hbm,
              feats_out, ast_out, pkf_v, pka_v, si_v, ti_v, hi_v,
              b_a, b_b, b_c, sem_a, sem_b, sem_c):
    c = lax.axis_index("c")
    s = lax.axis_index("s")
    wid = c * SC_SUBCORES + s
    pltpu.sync_copy(pkf_hbm.at[wid], pkf_v)
    pltpu.sync_copy(pka_hbm.at[wid], pka_v)

    @pl.loop(0, NF)
    def _feats(it):
        for j in range(KF // 16):
            sl = pl.ds(j * 16, 16)
            p = pkf_v[it, sl]
            hi_v[sl] = jnp.bitwise_and(p, jnp.full((16,), 31, jnp.int32))
            ti_v[sl] = jnp.bitwise_and(
                jnp.right_shift(p, jnp.full((16,), 5, jnp.int32)),
                jnp.full((16,), 255, jnp.int32))
            si_v[sl] = jnp.right_shift(p, jnp.full((16,), 13, jnp.int32))
        pltpu.async_copy(type_hbm.at[ti_v], b_a, sem_a)
        pltpu.async_copy(token_hbm.at[si_v], b_b, sem_b)
        pltpu.async_copy(height_hbm.at[hi_v], b_c, sem_c)
        pltpu.make_async_copy(type_hbm.at[ti_v], b_a, sem_a).wait()
        pltpu.make_async_copy(token_hbm.at[si_v], b_b, sem_b).wait()
        pltpu.make_async_copy(height_hbm.at[hi_v], b_c, sem_c).wait()
        _vadd3(b_a, b_b, b_c, KF)
        pltpu.sync_copy(b_a, feats_out.at[pl.ds(wid * NF * KF + it * KF, KF), :])

    @pl.loop(0, NA)
    def _ast(it):
        dst = b_b.at[pl.ds(0, KA), :]
        pltpu.async_copy(token_hbm.at[pka_v.at[it]], dst, sem_b)
        pltpu.make_async_copy(token_hbm.at[pka_v.at[it]], dst, sem_b).wait()
        pltpu.sync_copy(dst, ast_out.at[pl.ds(wid * NA * KA + it * KA, KA), :])


@functools.partial(
    pl.kernel,
    out_type=jax.ShapeDtypeStruct((N_PAD, D), jnp.float32),
    mesh=_sc_mesh,
    scratch_types=[
        pltpu.VMEM((NH, KH), jnp.int32),
        pltpu.VMEM((KH,), jnp.int32),
        pltpu.VMEM((KH,), jnp.int32),
        pltpu.VMEM((KH, D), jnp.float32),
        pltpu.VMEM((KH, D), jnp.float32),
        pltpu.SemaphoreType.DMA,
        pltpu.SemaphoreType.DMA,
    ],
)
def _sc_h0(emb_hbm, central_hbm, pkh_hbm, h0_out,
           pkh_v, ei_v, di_v, b_a, b_b, sem_a, sem_b):
    c = lax.axis_index("c")
    s = lax.axis_index("s")
    wid = c * SC_SUBCORES + s
    pltpu.sync_copy(pkh_hbm.at[wid], pkh_v)

    @pl.loop(0, NH)
    def _rows(it):
        for j in range(KH // 16):
            sl = pl.ds(j * 16, 16)
            p = pkh_v[it, sl]
            di_v[sl] = jnp.bitwise_and(p, jnp.full((16,), 255, jnp.int32))
            ei_v[sl] = jnp.right_shift(p, jnp.full((16,), 8, jnp.int32))
        pltpu.async_copy(emb_hbm.at[ei_v], b_a, sem_a)
        pltpu.async_copy(central_hbm.at[di_v], b_b, sem_b)
        pltpu.make_async_copy(emb_hbm.at[ei_v], b_a, sem_a).wait()
        pltpu.make_async_copy(central_hbm.at[di_v], b_b, sem_b).wait()

        @pl.loop(0, KH)
        def _add(r):
            for j in range(D // 16):
                sl = pl.ds(j * 16, 16)
                b_a[r, sl] = b_a[r, sl] + b_b[r, sl]

        pltpu.sync_copy(b_a, h0_out.at[pl.ds(wid * NH * KH + it * KH, KH), :])


def _pool_body(h_ref, ls_ref, wa_ref, seg_ref, fcw_ref, fcb_ref, out_ref):
    h = h_ref[...]
    ls = ls_ref[...]
    iota_n = jax.lax.broadcasted_iota(jnp.int32, (128, N), 1)
    onehot_ls = (ls == iota_n).astype(jnp.float32)
    hl = jnp.dot(onehot_ls, h, preferred_element_type=jnp.float32)
    ctx = jnp.dot(hl, wa_ref[...],
                  preferred_element_type=jnp.float32)
    seg = seg_ref[...]
    iota = jax.lax.broadcasted_iota(jnp.int32, (N, 128), 1)
    onehot = (seg == iota).astype(jnp.float32)
    ctx_rows = jnp.dot(onehot, ctx, preferred_element_type=jnp.float32)
    scores = jax.nn.sigmoid(
        jnp.sum(h * ctx_rows, axis=-1, keepdims=True))
    ge = jax.lax.dot_general(
        onehot, scores * h, (((0,), (0,)), ((), ())),
        preferred_element_type=jnp.float32)
    out_ref[...] = jnp.dot(ge, fcw_ref[...],
                           preferred_element_type=jnp.float32) + fcb_ref[...]


def _pool(h, ls_pad, w_attn, segment_ids, fc_w_pad, fc_b_pad):
    return pl.pallas_call(
        _pool_body,
        in_specs=[
            pl.BlockSpec((N, D), lambda: (0, 0)),
            pl.BlockSpec((128, 1), lambda: (0, 0)),
            pl.BlockSpec((D, D), lambda: (0, 0)),
            pl.BlockSpec((N, 1), lambda: (0, 0)),
            pl.BlockSpec((D, 128), lambda: (0, 0)),
            pl.BlockSpec((1, 128), lambda: (0, 0)),
        ],
        out_specs=pl.BlockSpec((128, 128), lambda: (0, 0)),
        out_shape=jax.ShapeDtypeStruct((128, 128), jnp.float32),
    )(h, ls_pad, w_attn, segment_ids.reshape(N, 1), fc_w_pad, fc_b_pad)


def kernel(token_table, type_table, height_table, W_t, W_l, b_conv,
           central_table, A, gru_W, gru_U, w_attn, fc_W, fc_b,
           ast_node_token_id, batch_node_type_id, batch_node_sub_tokens_id,
           batch_node_height, batch_children_index, ast_node_index,
           batch_tree_index, edge_index, edge_type, in_degrees, segment_ids,
           last_stmts):
    tid = batch_node_type_id.reshape(-1)
    sid = batch_node_sub_tokens_id.reshape(-1)
    hid = jnp.clip(batch_node_height, 0, 29).reshape(-1)
    pk_feats = ((sid * 256 + tid) * 32 + hid).reshape(NW, NF, KF)
    pk_ast = (jnp.zeros((AST_PAD,), jnp.int32).at[:N_AST]
              .set(ast_node_token_id).reshape(NW, NA, KA))
    feats_flat, ast_pad = _sc_embed(type_table, token_table, height_table,
                                    pk_feats, pk_ast)
    tree_emb = _tbcnn(feats_flat.reshape(N_TREE, T, D),
                      batch_children_index, W_t, W_l, b_conv)
    embeddings = jnp.concatenate([ast_pad[:N_AST], tree_emb], axis=0)
    permcat = jnp.concatenate([ast_node_index, batch_tree_index])
    invperm = jnp.zeros((N_PAD,), jnp.int32).at[permcat].set(
        jnp.arange(N, dtype=jnp.int32))
    degidx = jnp.zeros((N_PAD,), jnp.int32).at[:N].set(
        jnp.clip(in_degrees, 0, 149))
    pk_h0 = (invperm * 256 + degidx).reshape(NW, NH, KH)
    h = _sc_h0(embeddings, central_table, pk_h0)[:N]

    src, dst = edge_index[0], edge_index[1]
    packed = ((edge_type * N + src) * 16384 + dst).reshape(NW, NITER, K)
    layer_of_step = [l for l in range(len(TIME_STEPS))
                     for _ in range(TIME_STEPS[l])]
    n_steps = len(layer_of_step)
    tf = _transform(h, A[layer_of_step[0]])
    for i, l in enumerate(layer_of_step):
        agg2 = _edge_agg(tf.reshape(N_ETYPES * N, D), packed)
        do_ln = (i + 1 == n_steps) or (layer_of_step[i + 1] != l)
        with_tf = i + 1 < n_steps
        l_next = layer_of_step[min(i + 1, n_steps - 1)]
        res = _ggnn_step(h, agg2, gru_W[l], gru_U[l], A[l_next], do_ln, with_tf)
        if with_tf:
            h, tf = res
        else:
            h = res

    ls_pad = jnp.zeros((128, 1), jnp.int32).at[:G, 0].set(last_stmts)
    fc_w_pad = jnp.zeros((D, 128), jnp.float32).at[:, :N_CLASSES].set(fc_W)
    fc_b_pad = jnp.zeros((1, 128), jnp.float32).at[0, :N_CLASSES].set(fc_b)
    logits = _pool(h, ls_pad, w_attn, segment_ids, fc_w_pad, fc_b_pad)
    return (embeddings, logits[:G, :N_CLASSES])


# speedup vs baseline: 35.9884x; 1.0934x over previous
import functools

import jax
import jax.numpy as jnp
from jax import lax
from jax.experimental import pallas as pl
from jax.experimental.pallas import tpu as pltpu
from jax.experimental.pallas import tpu_sc as plsc

D = 128
N_AST = 8000
N_TREE = 2000
N = N_AST + N_TREE
T = 32
C = 4
E = 320000
N_ETYPES = 3
G = 100
N_CLASSES = 104
TIME_STEPS = [3, 3]

TB = 8
RB = 1000


def _tbcnn_body(feats_ref, child_ref, wt_ref, wl_ref, b_ref, out_ref):
    feats = feats_ref[...]
    child = child_ref[...]
    iota = jax.lax.broadcasted_iota(jnp.int32, (TB, T, C, T), 3)
    onehot = (child[..., None] == iota).astype(jnp.float32)
    p = jnp.sum(onehot, axis=2) * (1.0 / C)
    mean_child = jax.lax.dot_general(
        p, feats, (((2,), (1,)), ((0,), (0,))),
        preferred_element_type=jnp.float32)
    f2 = feats.reshape(TB * T, D)
    mc2 = mean_child.reshape(TB * T, D)
    conv = jnp.maximum(
        jnp.dot(f2, wt_ref[...], preferred_element_type=jnp.float32)
        + jnp.dot(mc2, wl_ref[...], preferred_element_type=jnp.float32)
        + b_ref[...], 0.0)
    out_ref[...] = jnp.max(conv.reshape(TB, T, D), axis=1)


def _tbcnn(feats, children, w_t, w_l, b_conv):
    return pl.pallas_call(
        _tbcnn_body,
        grid=(N_TREE // TB,),
        in_specs=[
            pl.BlockSpec((TB, T, D), lambda i: (i, 0, 0)),
            pl.BlockSpec((TB, T, C), lambda i: (i, 0, 0)),
            pl.BlockSpec((D, D), lambda i: (0, 0)),
            pl.BlockSpec((D, D), lambda i: (0, 0)),
            pl.BlockSpec((1, D), lambda i: (0, 0)),
        ],
        out_specs=pl.BlockSpec((TB, D), lambda i: (i, 0)),
        out_shape=jax.ShapeDtypeStruct((N_TREE, D), jnp.float32),
    )(feats, children, w_t, w_l, b_conv.reshape(1, D))


def _transform_body(h_ref, a_ref, tf_ref):
    h = h_ref[...]
    for e in range(N_ETYPES):
        tf_ref[e] = jnp.dot(h, a_ref[e], preferred_element_type=jnp.float32)


def _transform(h, a_l):
    return pl.pallas_call(
        _transform_body,
        grid=(N // RB,),
        in_specs=[
            pl.BlockSpec((RB, D), lambda i: (i, 0)),
            pl.BlockSpec((N_ETYPES, D, D), lambda i: (0, 0, 0)),
        ],
        out_specs=pl.BlockSpec((N_ETYPES, RB, D), lambda i: (0, i, 0)),
        out_shape=jax.ShapeDtypeStruct((N_ETYPES, N, D), jnp.float32),
    )(h, a_l)


def _step_body(do_ln, with_tf, h_ref, agg_ref, gw_ref, gu_ref, an_ref,
               hn_ref, tf_ref=None):
    h = h_ref[...]
    agg = agg_ref[0] + agg_ref[1]
    z = jax.nn.sigmoid(
        jnp.dot(agg, gw_ref[0], preferred_element_type=jnp.float32)
        + jnp.dot(h, gu_ref[0], preferred_element_type=jnp.float32))
    r = jax.nn.sigmoid(
        jnp.dot(agg, gw_ref[1], preferred_element_type=jnp.float32)
        + jnp.dot(h, gu_ref[1], preferred_element_type=jnp.float32))
    hh = jnp.tanh(
        jnp.dot(agg, gw_ref[2], preferred_element_type=jnp.float32)
        + jnp.dot(r * h, gu_ref[2], preferred_element_type=jnp.float32))
    hn = (1.0 - z) * h + z * hh
    if do_ln:
        m = jnp.mean(hn, axis=-1, keepdims=True)
        v = jnp.mean((hn - m) ** 2, axis=-1, keepdims=True)
        hn = (hn - m) * jax.lax.rsqrt(v + 1e-5)
    hn_ref[...] = hn
    if with_tf:
        for e in range(N_ETYPES):
            tf_ref[e] = jnp.dot(hn, an_ref[e], preferred_element_type=jnp.float32)


def _ggnn_step(h, agg, gw_l, gu_l, a_next, do_ln, with_tf):
    in_specs = [
        pl.BlockSpec((RB, D), lambda i: (i, 0)),
        pl.BlockSpec((SC_CORES, RB, D), lambda i: (0, i, 0)),
        pl.BlockSpec((N_ETYPES, D, D), lambda i: (0, 0, 0)),
        pl.BlockSpec((N_ETYPES, D, D), lambda i: (0, 0, 0)),
        pl.BlockSpec((N_ETYPES, D, D), lambda i: (0, 0, 0)),
    ]
    if with_tf:
        out_specs = [
            pl.BlockSpec((RB, D), lambda i: (i, 0)),
            pl.BlockSpec((N_ETYPES, RB, D), lambda i: (0, i, 0)),
        ]
        out_shape = [
            jax.ShapeDtypeStruct((N, D), jnp.float32),
            jax.ShapeDtypeStruct((N_ETYPES, N, D), jnp.float32),
        ]
    else:
        out_specs = pl.BlockSpec((RB, D), lambda i: (i, 0))
        out_shape = jax.ShapeDtypeStruct((N, D), jnp.float32)
    return pl.pallas_call(
        functools.partial(_step_body, do_ln, with_tf),
        grid=(N // RB,),
        in_specs=in_specs,
        out_specs=out_specs,
        out_shape=out_shape,
    )(h, agg, gw_l, gu_l, a_next)


SC_CORES = 2
SC_SUBCORES = 16
NW = SC_CORES * SC_SUBCORES
EPW = E // NW
K = 80
NITER = EPW // K
N_PAD = 10240
RPS = N_PAD // SC_SUBCORES

_sc_mesh = plsc.VectorSubcoreMesh(core_axis_name="c", subcore_axis_name="s")


@functools.partial(
    pl.kernel,
    out_type=jax.ShapeDtypeStruct((SC_CORES, N_PAD, D), jnp.float32),
    mesh=_sc_mesh,
    scratch_types=[
        pltpu.VMEM((NITER, K), jnp.int32),
        pltpu.VMEM((3, K), jnp.int32),
        pltpu.VMEM((3, K), jnp.int32),
        pltpu.VMEM((K, D), jnp.float32),
        pltpu.VMEM((K, D), jnp.float32),
        pltpu.VMEM((K, D), jnp.float32),
        pltpu.VMEM_SHARED((N_PAD, D), jnp.float32),
        pltpu.SemaphoreType.DMA,
        pltpu.SemaphoreType.DMA,
        pltpu.SemaphoreType.DMA,
        pltpu.SemaphoreType.DMA,
        pltpu.SemaphoreType.DMA,
        pltpu.SemaphoreType.DMA,
        pltpu.SemaphoreType.DMA,
    ],
)
def _edge_agg(tf_hbm, pk_hbm, out_hbm,
              pk_v, idxu, dstu, rows_a, rows_b, rows_c, agg_sh,
              sem_a, sem_b, sem_c, sem_sa, sem_sb, sem_sc, sem_i):
    c = lax.axis_index("c")
    s = lax.axis_index("s")
    wid = c * SC_SUBCORES + s
    rows = (rows_a, rows_b, rows_c)
    sems = (sem_a, sem_b, sem_c)
    ssems = (sem_sa, sem_sb, sem_sc)

    pltpu.async_copy(pk_hbm.at[wid], pk_v, sem_i)

    @pl.loop(0, K)
    def _zero_rows(i):
        for j in range(D // 16):
            rows_a[i, pl.ds(j * 16, 16)] = jnp.zeros((16,), jnp.float32)

    row0 = s * RPS
    for b in range(RPS // K):
        pltpu.sync_copy(rows_a, agg_sh.at[pl.ds(row0 + b * K, K), :])
    pltpu.make_async_copy(pk_hbm.at[wid], pk_v, sem_i).wait()
    plsc.subcore_barrier()

    def _unpack(cur, b):
        for j in range(K // 16):
            v = pk_v[cur, pl.ds(j * 16, 16)]
            idxu[b, pl.ds(j * 16, 16)] = jnp.right_shift(
                v, jnp.full((16,), 14, jnp.int32))
            dstu[b, pl.ds(j * 16, 16)] = jnp.bitwise_and(
                v, jnp.full((16,), 16383, jnp.int32))

    _unpack(0, 0)
    _unpack(1, 1)
    pltpu.async_copy(tf_hbm.at[idxu.at[0]], rows[0], sems[0])
    pltpu.async_copy(tf_hbm.at[idxu.at[1]], rows[1], sems[1])

    NFULL = 3 * ((NITER - 2) // 3)

    @pl.loop(0, NFULL, step=3)
    def _edges(it):
        for b in range(3):
            cur = it + b
            j = (b + 2) % 3
            pltpu.make_async_copy(tf_hbm.at[idxu.at[b]], rows[b],
                                  sems[b]).wait()
            pltpu.async_copy(rows[b], agg_sh.at[dstu.at[b]], ssems[b],
                             add=True)

            @pl.when(cur >= 1)
            def _drain():
                pltpu.make_async_copy(rows[j], agg_sh.at[dstu.at[j]],
                                      ssems[j]).wait()
            _unpack(cur + 2, j)
            pltpu.async_copy(tf_hbm.at[idxu.at[j]], rows[j], sems[j])

    for cur in range(NFULL, NITER):
        b = cur % 3
        pltpu.make_async_copy(tf_hbm.at[idxu.at[b]], rows[b], sems[b]).wait()
        pltpu.async_copy(rows[b], agg_sh.at[dstu.at[b]], ssems[b], add=True)
    for b in range(3):
        pltpu.make_async_copy(rows[b], agg_sh.at[dstu.at[b]], ssems[b]).wait()

    plsc.subcore_barrier()
    pltpu.sync_copy(agg_sh.at[pl.ds(row0, RPS), :],
                    out_hbm.at[c, pl.ds(row0, RPS), :])


KF, NF = 80, 25
KA, NA = 64, 4
KH, NH = 80, 4
AST_PAD = NW * KA * NA


def _vadd3(dst, a, b, nrows):
    @pl.loop(0, nrows)
    def _rows(r):
        for j in range(D // 16):
            sl = pl.ds(j * 16, 16)
            dst[r, sl] = dst[r, sl] + a[r, sl] + b[r, sl]


@functools.partial(
    pl.kernel,
    out_type=(jax.ShapeDtypeStruct((N_TREE * T, D), jnp.float32),
              jax.ShapeDtypeStruct((AST_PAD, D), jnp.float32)),
    mesh=_sc_mesh,
    scratch_types=[
        pltpu.VMEM((NF, KF), jnp.int32),
        pltpu.VMEM((NA, KA), jnp.int32),
        pltpu.VMEM((KF,), jnp.int32),
        pltpu.VMEM((KF,), jnp.int32),
        pltpu.VMEM((KF,), jnp.int32),
        pltpu.VMEM((KF, D), jnp.float32),
        pltpu.VMEM((KF, D), jnp.float32),
        pltpu.VMEM((KF, D), jnp.float32),
        pltpu.SemaphoreType.DMA,
        pltpu.SemaphoreType.DMA,
        pltpu.SemaphoreType.DMA,
    ],
)
def _sc_embed(type_hbm, token_hbm, height_hbm, pkf_hbm, pka_hbm,
              feats_out, ast_out, pkf_v, pka_v, si_v, ti_v, hi_v,
              b_a, b_b, b_c, sem_a, sem_b, sem_c):
    c = lax.axis_index("c")
    s = lax.axis_index("s")
    wid = c * SC_SUBCORES + s
    pltpu.sync_copy(pkf_hbm.at[wid], pkf_v)
    pltpu.sync_copy(pka_hbm.at[wid], pka_v)

    @pl.loop(0, NF)
    def _feats(it):
        for j in range(KF // 16):
            sl = pl.ds(j * 16, 16)
            p = pkf_v[it, sl]
            hi_v[sl] = jnp.bitwise_and(p, jnp.full((16,), 31, jnp.int32))
            ti_v[sl] = jnp.bitwise_and(
                jnp.right_shift(p, jnp.full((16,), 5, jnp.int32)),
                jnp.full((16,), 255, jnp.int32))
            si_v[sl] = jnp.right_shift(p, jnp.full((16,), 13, jnp.int32))
        pltpu.async_copy(type_hbm.at[ti_v], b_a, sem_a)
        pltpu.async_copy(token_hbm.at[si_v], b_b, sem_b)
        pltpu.async_copy(height_hbm.at[hi_v], b_c, sem_c)
        pltpu.make_async_copy(type_hbm.at[ti_v], b_a, sem_a).wait()
        pltpu.make_async_copy(token_hbm.at[si_v], b_b, sem_b).wait()
        pltpu.make_async_copy(height_hbm.at[hi_v], b_c, sem_c).wait()
        _vadd3(b_a, b_b, b_c, KF)
        pltpu.sync_copy(b_a, feats_out.at[pl.ds(wid * NF * KF + it * KF, KF), :])

    @pl.loop(0, NA)
    def _ast(it):
        dst = b_b.at[pl.ds(0, KA), :]
        pltpu.async_copy(token_hbm.at[pka_v.at[it]], dst, sem_b)
        pltpu.make_async_copy(token_hbm.at[pka_v.at[it]], dst, sem_b).wait()
        pltpu.sync_copy(dst, ast_out.at[pl.ds(wid * NA * KA + it * KA, KA), :])


@functools.partial(
    pl.kernel,
    out_type=jax.ShapeDtypeStruct((N_PAD, D), jnp.float32),
    mesh=_sc_mesh,
    scratch_types=[
        pltpu.VMEM((NH, KH), jnp.int32),
        pltpu.VMEM((KH,), jnp.int32),
        pltpu.VMEM((KH,), jnp.int32),
        pltpu.VMEM((KH, D), jnp.float32),
        pltpu.VMEM((KH, D), jnp.float32),
        pltpu.SemaphoreType.DMA,
        pltpu.SemaphoreType.DMA,
    ],
)
def _sc_h0(emb_hbm, central_hbm, pkh_hbm, h0_out,
           pkh_v, ei_v, di_v, b_a, b_b, sem_a, sem_b):
    c = lax.axis_index("c")
    s = lax.axis_index("s")
    wid = c * SC_SUBCORES + s
    pltpu.sync_copy(pkh_hbm.at[wid], pkh_v)

    @pl.loop(0, NH)
    def _rows(it):
        for j in range(KH // 16):
            sl = pl.ds(j * 16, 16)
            p = pkh_v[it, sl]
            di_v[sl] = jnp.bitwise_and(p, jnp.full((16,), 255, jnp.int32))
            ei_v[sl] = jnp.right_shift(p, jnp.full((16,), 8, jnp.int32))
        pltpu.async_copy(emb_hbm.at[ei_v], b_a, sem_a)
        pltpu.async_copy(central_hbm.at[di_v], b_b, sem_b)
        pltpu.make_async_copy(emb_hbm.at[ei_v], b_a, sem_a).wait()
        pltpu.make_async_copy(central_hbm.at[di_v], b_b, sem_b).wait()

        @pl.loop(0, KH)
        def _add(r):
            for j in range(D // 16):
                sl = pl.ds(j * 16, 16)
                b_a[r, sl] = b_a[r, sl] + b_b[r, sl]

        pltpu.sync_copy(b_a, h0_out.at[pl.ds(wid * NH * KH + it * KH, KH), :])


def _pool_body(h_ref, ls_ref, wa_ref, seg_ref, fcw_ref, fcb_ref, out_ref):
    h = h_ref[...]
    ls = ls_ref[...]
    iota_n = jax.lax.broadcasted_iota(jnp.int32, (128, N), 1)
    onehot_ls = (ls == iota_n).astype(jnp.float32)
    hl = jnp.dot(onehot_ls, h, preferred_element_type=jnp.float32)
    ctx = jnp.dot(hl, wa_ref[...],
                  preferred_element_type=jnp.float32)
    seg = seg_ref[...]
    iota = jax.lax.broadcasted_iota(jnp.int32, (N, 128), 1)
    onehot = (seg == iota).astype(jnp.float32)
    ctx_rows = jnp.dot(onehot, ctx, preferred_element_type=jnp.float32)
    scores = jax.nn.sigmoid(
        jnp.sum(h * ctx_rows, axis=-1, keepdims=True))
    ge = jax.lax.dot_general(
        onehot, scores * h, (((0,), (0,)), ((), ())),
        preferred_element_type=jnp.float32)
    out_ref[...] = jnp.dot(ge, fcw_ref[...],
                           preferred_element_type=jnp.float32) + fcb_ref[...]


def _pool(h, ls_pad, w_attn, segment_ids, fc_w_pad, fc_b_pad):
    return pl.pallas_call(
        _pool_body,
        in_specs=[
            pl.BlockSpec((N, D), lambda: (0, 0)),
            pl.BlockSpec((128, 1), lambda: (0, 0)),
            pl.BlockSpec((D, D), lambda: (0, 0)),
            pl.BlockSpec((N, 1), lambda: (0, 0)),
            pl.BlockSpec((D, 128), lambda: (0, 0)),
            pl.BlockSpec((1, 128), lambda: (0, 0)),
        ],
        out_specs=pl.BlockSpec((128, 128), lambda: (0, 0)),
        out_shape=jax.ShapeDtypeStruct((128, 128), jnp.float32),
    )(h, ls_pad, w_attn, segment_ids.reshape(N, 1), fc_w_pad, fc_b_pad)


def kernel(token_table, type_table, height_table, W_t, W_l, b_conv,
           central_table, A, gru_W, gru_U, w_attn, fc_W, fc_b,
           ast_node_token_id, batch_node_type_id, batch_node_sub_tokens_id,
           batch_node_height, batch_children_index, ast_node_index,
           batch_tree_index, edge_index, edge_type, in_degrees, segment_ids,
           last_stmts):
    tid = batch_node_type_id.reshape(-1)
    sid = batch_node_sub_tokens_id.reshape(-1)
    hid = jnp.clip(batch_node_height, 0, 29).reshape(-1)
    pk_feats = ((sid * 256 + tid) * 32 + hid).reshape(NW, NF, KF)
    pk_ast = (jnp.zeros((AST_PAD,), jnp.int32).at[:N_AST]
              .set(ast_node_token_id).reshape(NW, NA, KA))
    feats_flat, ast_pad = _sc_embed(type_table, token_table, height_table,
                                    pk_feats, pk_ast)
    tree_emb = _tbcnn(feats_flat.reshape(N_TREE, T, D),
                      batch_children_index, W_t, W_l, b_conv)
    embeddings = jnp.concatenate([ast_pad[:N_AST], tree_emb], axis=0)
    permcat = jnp.concatenate([ast_node_index, batch_tree_index])
    invperm = jnp.zeros((N_PAD,), jnp.int32).at[permcat].set(
        jnp.arange(N, dtype=jnp.int32))
    degidx = jnp.zeros((N_PAD,), jnp.int32).at[:N].set(
        jnp.clip(in_degrees, 0, 149))
    pk_h0 = (invperm * 256 + degidx).reshape(NW, NH, KH)
    h = _sc_h0(embeddings, central_table, pk_h0)[:N]

    src, dst = edge_index[0], edge_index[1]
    packed = ((edge_type * N + src) * 16384 + dst).reshape(NW, NITER, K)
    layer_of_step = [l for l in range(len(TIME_STEPS))
                     for _ in range(TIME_STEPS[l])]
    n_steps = len(layer_of_step)
    tf = _transform(h, A[layer_of_step[0]])
    for i, l in enumerate(layer_of_step):
        agg2 = _edge_agg(tf.reshape(N_ETYPES * N, D), packed)
        do_ln = (i + 1 == n_steps) or (layer_of_step[i + 1] != l)
        with_tf = i + 1 < n_steps
        l_next = layer_of_step[min(i + 1, n_steps - 1)]
        res = _ggnn_step(h, agg2, gru_W[l], gru_U[l], A[l_next], do_ln, with_tf)
        if with_tf:
            h, tf = res
        else:
            h = res

    ls_pad = jnp.zeros((128, 1), jnp.int32).at[:G, 0].set(last_stmts)
    fc_w_pad = jnp.zeros((D, 128), jnp.float32).at[:, :N_CLASSES].set(fc_W)
    fc_b_pad = jnp.zeros((1, 128), jnp.float32).at[0, :N_CLASSES].set(fc_b)
    logits = _pool(h, ls_pad, w_attn, segment_ids, fc_w_pad, fc_b_pad)
    return (embeddings, logits[:G, :N_CLASSES])
